# Initial kernel scaffold; baseline (speedup 1.0000x reference)
import jax
import jax.numpy as jnp
from jax.experimental import pallas as pl


def kernel(p, x, n_p, W, gamma, beta, o, n_o, knn_idx):
    raise NotImplementedError("write your pallas kernel here")


# speedup vs baseline: 4.1755x; 4.1755x over previous
import functools

import jax
import jax.numpy as jnp
from jax import lax
from jax.experimental import pallas as pl
from jax.experimental.pallas import tpu as pltpu
from jax.experimental.pallas import tpu_sc as plsc

_EPS = 1e-5


def _ln_proj_body(x_ref, w_ref, b_ref, y_ref):
    x = x_ref[...]
    mean = jnp.mean(x, axis=1, keepdims=True)
    xc = x - mean
    var = jnp.mean(xc * xc, axis=1, keepdims=True)
    normed = xc * lax.rsqrt(var + _EPS)
    y_ref[...] = (
        jnp.dot(normed, w_ref[...], preferred_element_type=jnp.float32)
        + b_ref[...]
    )


def _ln_proj(x, w2, b2, block_rows=2000):
    n, c = x.shape
    out = w2.shape[1]
    grid = n // block_rows
    return pl.pallas_call(
        _ln_proj_body,
        grid=(grid,),
        in_specs=[
            pl.BlockSpec((block_rows, c), lambda i: (i, 0)),
            pl.BlockSpec((c, out), lambda i: (0, 0)),
            pl.BlockSpec((1, out), lambda i: (0, 0)),
        ],
        out_specs=pl.BlockSpec((block_rows, out), lambda i: (i, 0)),
        out_shape=jax.ShapeDtypeStruct((n, out), jnp.float32),
    )(x, w2, b2)


def _make_gather_max(m_pad, k, d, cb):
    info = plsc.get_sparse_core_info()
    nc, ns, lanes = info.num_cores, info.num_subcores, info.num_lanes
    nw = nc * ns
    m_per_w = m_pad // nw
    n_chunks = m_per_w // cb
    assert m_per_w % cb == 0
    mesh = plsc.VectorSubcoreMesh(core_axis_name="c", subcore_axis_name="s")

    @functools.partial(
        pl.kernel,
        mesh=mesh,
        out_type=jax.ShapeDtypeStruct((m_pad, d), jnp.float32),
        scratch_types=[
            pltpu.VMEM((m_per_w * k,), jnp.int32),
            pltpu.VMEM((cb * k, d), jnp.float32),
            pltpu.VMEM((cb, d), jnp.float32),
            pltpu.SemaphoreType.DMA,
        ],
    )
    def gm(table_hbm, idx_hbm, out_hbm, idx_v, rows_v, out_v, sem):
        wid = lax.axis_index("s") * nc + lax.axis_index("c")
        ibase = wid * (m_per_w * k)
        pltpu.sync_copy(idx_hbm.at[pl.ds(ibase, m_per_w * k)], idx_v)

        def chunk_body(g, carry):
            pltpu.async_copy(
                table_hbm.at[idx_v.at[pl.ds(g * (cb * k), cb * k)]],
                rows_v,
                sem,
            ).wait()

            def center_body(i, carry2):
                for c in range(d // lanes):
                    acc = rows_v[i * k, pl.ds(c * lanes, lanes)]
                    for r in range(1, k):
                        acc = jnp.maximum(
                            acc, rows_v[i * k + r, pl.ds(c * lanes, lanes)]
                        )
                    out_v[i, pl.ds(c * lanes, lanes)] = acc
                return carry2

            lax.fori_loop(0, cb, center_body, 0, unroll=False)
            pltpu.sync_copy(
                out_v, out_hbm.at[pl.ds(wid * m_per_w + g * cb, cb)]
            )
            return carry

        lax.fori_loop(0, n_chunks, chunk_body, 0, unroll=False)

    return gm


def kernel(p, x, n_p, W, gamma, beta, o, n_o, knn_idx):
    m, k = knn_idx.shape
    c = x.shape[1]
    out = W.shape[0]

    w2 = (W * gamma[None, :]).T
    b2 = (beta @ W.T)[None, :]

    y = _ln_proj(x, w2, b2)

    nw = 32
    cb = 16
    m_pad = ((m + nw * cb - 1) // (nw * cb)) * (nw * cb)
    idx_flat = jnp.pad(knn_idx, ((0, m_pad - m), (0, 0))).reshape(-1)

    feats = _make_gather_max(m_pad, k, out, cb)(y, idx_flat)[:m]
    return (feats, n_p, n_o)


# speedup vs baseline: 5.2769x; 1.2638x over previous
import functools

import jax
import jax.numpy as jnp
from jax import lax
from jax.experimental import pallas as pl
from jax.experimental.pallas import tpu as pltpu
from jax.experimental.pallas import tpu_sc as plsc

_EPS = 1e-5


def _ln_proj_body(x_ref, w_ref, b_ref, y_ref):
    x = x_ref[...]
    mean = jnp.mean(x, axis=1, keepdims=True)
    xc = x - mean
    var = jnp.mean(xc * xc, axis=1, keepdims=True)
    normed = xc * lax.rsqrt(var + _EPS)
    y_ref[...] = (
        jnp.dot(normed, w_ref[...], preferred_element_type=jnp.float32)
        + b_ref[...]
    )


def _ln_proj(x, w2, b2, block_rows=2000):
    n, c = x.shape
    out = w2.shape[1]
    grid = n // block_rows
    return pl.pallas_call(
        _ln_proj_body,
        grid=(grid,),
        in_specs=[
            pl.BlockSpec((block_rows, c), lambda i: (i, 0)),
            pl.BlockSpec((c, out), lambda i: (0, 0)),
            pl.BlockSpec((1, out), lambda i: (0, 0)),
        ],
        out_specs=pl.BlockSpec((block_rows, out), lambda i: (i, 0)),
        out_shape=jax.ShapeDtypeStruct((n, out), jnp.float32),
    )(x, w2, b2)


def _make_gather_max(m_pad, k, d, cb):
    info = plsc.get_sparse_core_info()
    nc, ns, lanes = info.num_cores, info.num_subcores, info.num_lanes
    nw = nc * ns
    m_per_w = m_pad // nw
    n_chunks = m_per_w // cb
    assert m_per_w % cb == 0
    mesh = plsc.VectorSubcoreMesh(core_axis_name="c", subcore_axis_name="s")

    @functools.partial(
        pl.kernel,
        mesh=mesh,
        out_type=jax.ShapeDtypeStruct((m_pad, d), jnp.float32),
        scratch_types=[
            pltpu.VMEM((m_per_w * k,), jnp.int32),
            pltpu.VMEM((cb * k, d), jnp.float32),
            pltpu.VMEM((cb * k, d), jnp.float32),
            pltpu.VMEM((cb, d), jnp.float32),
            pltpu.SemaphoreType.DMA,
            pltpu.SemaphoreType.DMA,
        ],
    )
    def gm(table_hbm, idx_hbm, out_hbm, idx_v, rows0_v, rows1_v, out_v,
           sem0, sem1):
        wid = lax.axis_index("s") * nc + lax.axis_index("c")
        ibase = wid * (m_per_w * k)
        pltpu.sync_copy(idx_hbm.at[pl.ds(ibase, m_per_w * k)], idx_v)

        rows = (rows0_v, rows1_v)
        sems = (sem0, sem1)

        def start(g, buf):
            pltpu.async_copy(
                table_hbm.at[idx_v.at[pl.ds(g * (cb * k), cb * k)]],
                rows[buf],
                sems[buf],
            )

        def wait(buf):
            pltpu.make_async_copy(
                table_hbm.at[idx_v.at[pl.ds(0, cb * k)]],
                rows[buf],
                sems[buf],
            ).wait()

        def compute(g, buf):
            rows_v = rows[buf]

            def center_body(i, carry2):
                for c in range(d // lanes):
                    acc = rows_v[i * k, pl.ds(c * lanes, lanes)]
                    for r in range(1, k):
                        acc = jnp.maximum(
                            acc, rows_v[i * k + r, pl.ds(c * lanes, lanes)]
                        )
                    out_v[i, pl.ds(c * lanes, lanes)] = acc
                return carry2

            lax.fori_loop(0, cb, center_body, 0, unroll=False)
            pltpu.sync_copy(
                out_v, out_hbm.at[pl.ds(wid * m_per_w + g * cb, cb)]
            )

        start(0, 0)

        def pair_body(gp, carry):
            g0 = 2 * gp

            @pl.when(g0 + 1 < n_chunks)
            def _():
                start(g0 + 1, 1)

            wait(0)
            compute(g0, 0)

            @pl.when(g0 + 2 < n_chunks)
            def _():
                start(g0 + 2, 0)

            @pl.when(g0 + 1 < n_chunks)
            def _():
                wait(1)
                compute(g0 + 1, 1)

            return carry

        lax.fori_loop(0, (n_chunks + 1) // 2, pair_body, 0, unroll=False)

    return gm


def kernel(p, x, n_p, W, gamma, beta, o, n_o, knn_idx):
    m, k = knn_idx.shape
    c = x.shape[1]
    out = W.shape[0]

    w2 = (W * gamma[None, :]).T
    b2 = (beta @ W.T)[None, :]

    y = _ln_proj(x, w2, b2)

    nw = 32
    cb = 16
    m_pad = ((m + nw * cb - 1) // (nw * cb)) * (nw * cb)
    idx_flat = jnp.pad(knn_idx, ((0, m_pad - m), (0, 0))).reshape(-1)

    feats = _make_gather_max(m_pad, k, out, cb)(y, idx_flat)[:m]
    return (feats, n_p, n_o)
